# baseline (speedup 1.0000x reference)
import functools

import jax
import jax.numpy as jnp
from jax import lax
from jax.experimental import pallas as pl
from jax.experimental.pallas import tpu as pltpu, tpu_sc as plsc

_NT = 16
_NB2 = 512
_NB3 = 63
_TBL2 = _NB2 * _NB2
_TBL3 = _NB3 ** 3
_TBL3P = 250112
_NW = 32
_C = 1024
_SUB = _C // 128


def _wid():
    return lax.axis_index("s") * 2 + lax.axis_index("c")


def _red16(g, src2, dst2, do_clip):
    def body(sub, c):
        for j in range(8):
            ns = pl.ds(sub * 128 + j * 16, 16)
            acc = g[0, ns]
            for t in range(1, _NT):
                acc = acc + g[t, ns]
            if do_clip:
                dst2[ns] = jnp.clip(src2[ns] + acc, 0.0, 1.0)
            else:
                dst2[ns] = acc
        return c
    lax.fori_loop(0, _SUB, body, 0)


def _k1_body(p_hbm, v_hbm, wp, wr, off_hbm, out,
             p_v, v_v, pp_v, rp_v, idx2, gp, gr, off_v, sem1, sem2, semo,
             *, n):
    npw = n // _NW
    nchunks = npw // _C
    w0 = _wid() * npw
    pltpu.sync_copy(off_hbm, off_v)

    def drain(s, cnt):
        def w(j, c):
            pltpu.make_async_copy(wp.at[pl.ds(0, 128)], gp.at[0, 0, pl.ds(0, 128)], s).wait()
            return c
        lax.fori_loop(0, cnt, w, 0)

    def draino(k):
        for _ in range(k):
            pltpu.make_async_copy(p_hbm.at[pl.ds(0, _C)], pp_v.at[0], semo).wait()

    def load_idx2_fire(ci, par):
        base = pl.multiple_of(w0 + ci * _C, _C)
        pltpu.sync_copy(p_hbm.at[pl.ds(base, _C)], p_v.at[par])
        pltpu.sync_copy(v_hbm.at[pl.ds(base, _C)], v_v.at[par])
        off2 = off_v[...]

        def body(k, c):
            s16 = pl.ds(k * 16, 16)
            up = p_v[par, s16] * jnp.float32(_NB2)
            uv = v_v[par, s16] * jnp.float32(_NB2)
            for t in range(_NT):
                sh = jnp.float32(t / _NT)
                ip = jnp.minimum((up + sh).astype(jnp.int32), _NB2 - 1)
                iv = jnp.minimum((uv + sh).astype(jnp.int32), _NB2 - 1)
                tconst = (t >> 3) * (8 * _TBL2) + (t & 7) * 128
                idx2[par, t, s16] = (
                    ((ip >> 7) << 10) + (iv << 12) + (ip & 127) + (off2 + tconst))
            return c
        lax.fori_loop(0, _C // 16, body, 0)

        def f(j, c):
            t = j // _SUB
            sl = pl.ds((j % _SUB) * 128, 128)
            pltpu.async_copy(wp.at[idx2.at[par, t, sl]], gp.at[par, t, sl], sem1)
            pltpu.async_copy(wr.at[idx2.at[par, t, sl]], gr.at[par, t, sl], sem2)
            return c
        lax.fori_loop(0, _NT * _SUB, f, 0)

    load_idx2_fire(0, 0)

    def chunk(ci, carry):
        par = ci & 1
        drain(sem1, _NT * _SUB)

        @pl.when(ci + 1 < nchunks)
        def _():
            load_idx2_fire(ci + 1, 1 - par)

        @pl.when(ci > 1)
        def _():
            draino(2)

        base = pl.multiple_of(w0 + ci * _C, _C)
        _red16(gp.at[par], p_v.at[par], pp_v.at[par], True)
        pltpu.async_copy(pp_v.at[par], out.at[pl.ds(base, _C)], semo)
        drain(sem2, _NT * _SUB)
        _red16(gr.at[par], None, rp_v.at[par], False)
        pltpu.async_copy(rp_v.at[par], out.at[pl.ds(n + base, _C)], semo)
        return carry

    lax.fori_loop(0, nchunks, chunk, 0)
    draino(4)


def _k2_body(p_hbm, v_hbm, ppr_hbm, wv, off_hbm, out,
             p_v, v_v, pp_v, vp_v, idx3, gv, off_v, semv, semo, *, n):
    npw = n // _NW
    nchunks = npw // _C
    w0 = _wid() * npw
    pltpu.sync_copy(off_hbm, off_v)

    def drain(s, cnt):
        def w(j, c):
            pltpu.make_async_copy(wv.at[pl.ds(0, 128)], gv.at[0, 0, pl.ds(0, 128)], s).wait()
            return c
        lax.fori_loop(0, cnt, w, 0)

    def load_idx3_fire(ci, par):
        base = pl.multiple_of(w0 + ci * _C, _C)
        pltpu.sync_copy(p_hbm.at[pl.ds(base, _C)], p_v.at[par])
        pltpu.sync_copy(v_hbm.at[pl.ds(base, _C)], v_v.at[par])
        pltpu.sync_copy(ppr_hbm.at[pl.ds(base, _C)], pp_v.at[par])
        off3 = off_v[...]

        def body(k, c):
            s16 = pl.ds(k * 16, 16)
            u0 = p_v[par, s16] * jnp.float32(_NB3)
            u1 = v_v[par, s16] * jnp.float32(_NB3)
            u2 = pp_v[par, s16] * jnp.float32(_NB3)
            for t in range(_NT):
                sh = jnp.float32(t / _NT)
                i0 = jnp.minimum((u0 + sh).astype(jnp.int32), _NB3 - 1)
                i1 = jnp.minimum((u1 + sh).astype(jnp.int32), _NB3 - 1)
                i2 = jnp.minimum((u2 + sh).astype(jnp.int32), _NB3 - 1)
                idx3[par, t, s16] = (
                    i0 + i1 * _NB3 + i2 * (_NB3 * _NB3) + (off3 + t * _TBL3P))
            return c
        lax.fori_loop(0, _C // 16, body, 0)

        def f(j, c):
            t = j // _SUB
            sl = pl.ds((j % _SUB) * 128, 128)
            pltpu.async_copy(wv.at[idx3.at[par, t, sl]], gv.at[par, t, sl], semv)
            return c
        lax.fori_loop(0, _NT * _SUB, f, 0)

    load_idx3_fire(0, 0)

    def chunk(ci, carry):
        par = ci & 1
        drain(semv, _NT * _SUB)

        @pl.when(ci + 1 < nchunks)
        def _():
            load_idx3_fire(ci + 1, 1 - par)

        @pl.when(ci > 1)
        def _():
            pltpu.make_async_copy(p_hbm.at[pl.ds(0, _C)], vp_v.at[0], semo).wait()

        base = pl.multiple_of(w0 + ci * _C, _C)
        _red16(gv.at[par], v_v.at[par], vp_v.at[par], True)
        pltpu.async_copy(vp_v.at[par], out.at[pl.ds(base, _C)], semo)
        return carry

    lax.fori_loop(0, nchunks, chunk, 0)
    for _ in range(2):
        pltpu.make_async_copy(p_hbm.at[pl.ds(0, _C)], vp_v.at[0], semo).wait()


def kernel(state, W_p, W_v, W_r, action):
    n = state.shape[0]
    p_in = state[:, 0]
    v_in = state[:, 1]

    def _phys(W):
        na, nt, nf = W.shape
        return W.reshape(na, nt // 8, 8, nf // 128, 128).transpose(0, 1, 3, 2, 4).reshape(-1)

    wp = _phys(W_p)
    wv = jnp.pad(W_v, ((0, 0), (0, 0), (0, _TBL3P - W_v.shape[2]))).reshape(-1)
    wr = _phys(W_r)
    a = jnp.clip(jnp.asarray(action, jnp.int32), 0, W_p.shape[0] - 1)
    off2 = jnp.full((16,), a * (_NT * _TBL2), dtype=jnp.int32)
    off3 = jnp.full((16,), a * (_NT * _TBL3P), dtype=jnp.int32)

    mesh = plsc.VectorSubcoreMesh(core_axis_name="c", subcore_axis_name="s")
    ppr = pl.kernel(
        functools.partial(_k1_body, n=n),
        out_type=jax.ShapeDtypeStruct((2 * n,), jnp.float32),
        mesh=mesh,
        scratch_types=[
            pltpu.VMEM((2, _C), jnp.float32),
            pltpu.VMEM((2, _C), jnp.float32),
            pltpu.VMEM((2, _C), jnp.float32),
            pltpu.VMEM((2, _C), jnp.float32),
            pltpu.VMEM((2, _NT, _C), jnp.int32),
            pltpu.VMEM((2, _NT, _C), jnp.float32),
            pltpu.VMEM((2, _NT, _C), jnp.float32),
            pltpu.VMEM((16,), jnp.int32),
            pltpu.SemaphoreType.DMA,
            pltpu.SemaphoreType.DMA,
            pltpu.SemaphoreType.DMA,
        ],
    )(p_in, v_in, wp, wr, off2)

    vpr = pl.kernel(
        functools.partial(_k2_body, n=n),
        out_type=jax.ShapeDtypeStruct((n,), jnp.float32),
        mesh=mesh,
        scratch_types=[
            pltpu.VMEM((2, _C), jnp.float32),
            pltpu.VMEM((2, _C), jnp.float32),
            pltpu.VMEM((2, _C), jnp.float32),
            pltpu.VMEM((2, _C), jnp.float32),
            pltpu.VMEM((2, _NT, _C), jnp.int32),
            pltpu.VMEM((2, _NT, _C), jnp.float32),
            pltpu.VMEM((16,), jnp.int32),
            pltpu.SemaphoreType.DMA,
            pltpu.SemaphoreType.DMA,
        ],
    )(p_in, v_in, ppr, wv, off3)

    return jnp.stack([ppr[:n], vpr, ppr[n:]], axis=1)
